# Initial kernel scaffold; baseline (speedup 1.0000x reference)
import jax
import jax.numpy as jnp
from jax.experimental import pallas as pl


def kernel(new_keys, new_values, trainable_keys, trainable_values, mem_keys, mem_values, new_seq_ids):
    raise NotImplementedError("write your pallas kernel here")


# speedup vs baseline: 21.1245x; 21.1245x over previous
import jax
import jax.numpy as jnp
from jax.experimental import pallas as pl

N_HEADS = 16
HEAD_DIM = 128
N_TRAIN = 2048
S_NEW = 8192
CHUNK = 2048
S_OUT = N_TRAIN + S_NEW
N_CHUNKS = S_OUT // CHUNK


def _assemble_kernel(tk_ref, tv_ref, nk_ref, nv_ref, ok_ref, ov_ref):
    s = pl.program_id(1)

    @pl.when(s == 0)
    def _():
        ok_ref[...] = tk_ref[...]
        ov_ref[...] = tv_ref[...]

    @pl.when(s != 0)
    def _():
        ok_ref[...] = nk_ref[...]
        ov_ref[...] = nv_ref[...]


def kernel(new_keys, new_values, trainable_keys, trainable_values,
           mem_keys, mem_values, new_seq_ids):
    del mem_keys, mem_values, new_seq_ids

    blk = (1, 1, CHUNK, HEAD_DIM)
    train_spec = pl.BlockSpec(blk, lambda h, s: (0, h, 0, 0))
    new_spec = pl.BlockSpec(blk, lambda h, s: (0, h, jnp.maximum(s - 1, 0), 0))
    out_spec = pl.BlockSpec(blk, lambda h, s: (0, h, s, 0))

    out_shape = jax.ShapeDtypeStruct((1, N_HEADS, S_OUT, HEAD_DIM), jnp.float32)
    out_k, out_v = pl.pallas_call(
        _assemble_kernel,
        grid=(N_HEADS, N_CHUNKS),
        in_specs=[train_spec, train_spec, new_spec, new_spec],
        out_specs=[out_spec, out_spec],
        out_shape=[out_shape, out_shape],
    )(trainable_keys, trainable_values, new_keys, new_values)
    return out_k, out_v


# speedup vs baseline: 26.7334x; 1.2655x over previous
import jax
import jax.numpy as jnp
from jax.experimental import pallas as pl

N_HEADS = 16
HEAD_DIM = 128
N_TRAIN = 2048
S_NEW = 8192
S_OUT = N_TRAIN + S_NEW


def _assemble_kernel(tk_ref, tv_ref, nk_ref, nv_ref, ok_ref, ov_ref):
    ok_ref[0, 0, :N_TRAIN, :] = tk_ref[0, 0]
    ok_ref[0, 0, N_TRAIN:, :] = nk_ref[0, 0]
    ov_ref[0, 0, :N_TRAIN, :] = tv_ref[0, 0]
    ov_ref[0, 0, N_TRAIN:, :] = nv_ref[0, 0]


def kernel(new_keys, new_values, trainable_keys, trainable_values,
           mem_keys, mem_values, new_seq_ids):
    del mem_keys, mem_values, new_seq_ids

    train_spec = pl.BlockSpec((1, 1, N_TRAIN, HEAD_DIM), lambda h: (0, h, 0, 0))
    new_spec = pl.BlockSpec((1, 1, S_NEW, HEAD_DIM), lambda h: (0, h, 0, 0))
    out_spec = pl.BlockSpec((1, 1, S_OUT, HEAD_DIM), lambda h: (0, h, 0, 0))

    out_shape = jax.ShapeDtypeStruct((1, N_HEADS, S_OUT, HEAD_DIM), jnp.float32)
    out_k, out_v = pl.pallas_call(
        _assemble_kernel,
        grid=(N_HEADS,),
        in_specs=[train_spec, train_spec, new_spec, new_spec],
        out_specs=[out_spec, out_spec],
        out_shape=[out_shape, out_shape],
    )(trainable_keys, trainable_values, new_keys, new_values)
    return out_k, out_v
